# baseline (speedup 1.0000x reference)
import jax
import jax.numpy as jnp
from jax import lax
from jax.experimental import pallas as pl
from jax.experimental.pallas import tpu as pltpu
from jax.experimental.pallas import tpu_sc as plsc

DIM = 64
SCALE = 8.0
LANES = 16

NUM_CORES = 2
NUM_SUBCORES = 16
NUM_WORKERS = NUM_CORES * NUM_SUBCORES

JCHUNK = 2
NGBUF = 4
NOBUF = 2


def _make_body(n_xrows: int, xrow_len: int):
    rows_per_w = n_xrows // NUM_WORKERS
    idx_per_w = rows_per_w * xrow_len
    chunk_idx = JCHUNK * rows_per_w
    n_chunks = xrow_len // JCHUNK
    rshift = rows_per_w.bit_length() - 1

    def body(xp_hbm, table_hbm, out_hbm, idx_v,
             gbuf0, gbuf1, gbuf2, gbuf3, obuf0, obuf1,
             gsem0, gsem1, gsem2, gsem3, wsem0, wsem1):
        gbufs = (gbuf0, gbuf1, gbuf2, gbuf3)
        obufs = (obuf0, obuf1)
        gsems = (gsem0, gsem1, gsem2, gsem3)
        wsems = (wsem0, wsem1)
        wid = lax.axis_index("s") * NUM_CORES + lax.axis_index("c")
        base = wid * idx_per_w
        col0 = wid * rows_per_w
        pltpu.sync_copy(xp_hbm.at[pl.ds(base, idx_per_w)], idx_v)

        lane = lax.iota(jnp.int32, LANES)

        def fire_gather(g, b):
            pltpu.async_copy(
                table_hbm.at[idx_v.at[pl.ds(g * chunk_idx, chunk_idx)]],
                gbufs[b], gsems[b])

        def wait_gather(b):
            pltpu.make_async_copy(
                table_hbm.at[idx_v.at[pl.ds(0, chunk_idx)]],
                gbufs[b], gsems[b]).wait()

        def fire_write(g, b):
            pltpu.async_copy(
                obufs[b].at[(slice(None), pl.ds(0, rows_per_w))],
                out_hbm.at[(pl.ds(g * JCHUNK * DIM, JCHUNK * DIM),
                            pl.ds(col0, rows_per_w))],
                wsems[b])

        def wait_write(b):
            pltpu.make_async_copy(
                obufs[b].at[(slice(None), pl.ds(0, rows_per_w))],
                out_hbm.at[(pl.ds(0, JCHUNK * DIM), pl.ds(0, rows_per_w))],
                wsems[b]).wait()

        def transpose_scale(gb, ob):
            gbuf = gbufs[gb]
            obuf = obufs[ob]

            @pl.loop(0, chunk_idx, unroll=8)
            def _t(r, gbuf=gbuf, obuf=obuf):
                j = lax.shift_right_logical(r, rshift)
                i = lax.bitwise_and(r, rows_per_w - 1)
                iv = jnp.full((LANES,), i, dtype=jnp.int32)
                for kb in range(DIM // LANES):
                    vals = gbuf[r, pl.ds(LANES * kb, LANES)]
                    rowv = j * DIM + LANES * kb + lane
                    plsc.store_scatter(obuf, [rowv, iv], vals * SCALE)

        for g in range(NGBUF - 1):
            fire_gather(g, g)

        @pl.loop(0, n_chunks, step=NGBUF)
        def _chunks(gg):
            for b in range(NGBUF):
                g = gg + b
                ob = b % NOBUF
                wait_gather(b)

                @pl.when(g >= NOBUF)
                def _():
                    wait_write(ob)

                transpose_scale(b, ob)
                fire_write(g, ob)

                @pl.when(g + NGBUF - 1 < n_chunks)
                def _():
                    fire_gather(g + NGBUF - 1, (b + NGBUF - 1) % NGBUF)

        wait_write(0)
        wait_write(1)

    return body


def kernel(x, table):
    n_xrows, xrow_len = x.shape
    rows_per_w = n_xrows // NUM_WORKERS
    chunk_idx = JCHUNK * rows_per_w
    xp = jnp.transpose(
        x.reshape(NUM_WORKERS, rows_per_w, xrow_len), (0, 2, 1)).reshape(-1)

    mesh = plsc.VectorSubcoreMesh(core_axis_name="c", subcore_axis_name="s")
    out2d = pl.kernel(
        _make_body(n_xrows, xrow_len),
        out_type=jax.ShapeDtypeStruct((xrow_len * DIM, n_xrows), jnp.float32),
        mesh=mesh,
        compiler_params=pltpu.CompilerParams(
            use_tc_tiling_on_sc=False, needs_layout_passes=False,
            disable_bounds_checks=True),
        scratch_types=(
            [pltpu.VMEM((xp.size // NUM_WORKERS,), jnp.int32)]
            + [pltpu.VMEM((chunk_idx, DIM), jnp.float32)] * NGBUF
            + [pltpu.VMEM((JCHUNK * DIM, rows_per_w + 1), jnp.float32)] * NOBUF
            + [pltpu.SemaphoreType.DMA] * (NGBUF + NOBUF)
        ),
    )(xp, table)
    return out2d.reshape(xrow_len, DIM, n_xrows).transpose(2, 0, 1)
